# baseline (speedup 1.0000x reference)
import functools

import jax
import jax.numpy as jnp
from jax import lax
from jax.experimental import pallas as pl
from jax.experimental.pallas import tpu as pltpu
from jax.experimental.pallas import tpu_sc as plsc

N = 10000
E = 160000
F = 256
FH = 128
CLASSES = 2
DEGW = 16

NSUB = 16
NCORE = 2
NPAD = 10240
ROWS_PER_SUB = NPAD // NSUB
AGG_CHUNK = 80
AGG_NCH = E // (NSUB * AGG_CHUNK)
E_SPLITS = ((64000, 80), (96000, 40))

RN = 1024
RE = 2000


def _ln(y, g, b, eps=1e-5):
    m = jnp.mean(y, axis=-1, keepdims=True)
    v = jnp.mean((y - m) ** 2, axis=-1, keepdims=True)
    return (y - m) * lax.rsqrt(v + eps) * g + b


def _proj_body(h_ref, W_ref, b_ref, g_ref, bb_ref, xlo_ref, xhi_ref):
    y0 = jnp.dot(h_ref[:, :FH], W_ref[0], preferred_element_type=jnp.float32) + b_ref[0]
    y1 = jnp.dot(h_ref[:, FH:], W_ref[1], preferred_element_type=jnp.float32) + b_ref[1]
    xlo_ref[...] = jax.nn.relu(_ln(y0, g_ref[0], bb_ref[0]))
    xhi_ref[...] = jax.nn.relu(_ln(y1, g_ref[1], bb_ref[1]))


def _proj(h, proj_W, proj_b, proj_ln_g, proj_ln_b):
    return pl.pallas_call(
        _proj_body,
        grid=(NPAD // RN,),
        in_specs=[
            pl.BlockSpec((RN, F), lambda i: (i, 0)),
            pl.BlockSpec((2, FH, FH), lambda i: (0, 0, 0)),
            pl.BlockSpec((2, FH), lambda i: (0, 0)),
            pl.BlockSpec((2, FH), lambda i: (0, 0)),
            pl.BlockSpec((2, FH), lambda i: (0, 0)),
        ],
        out_specs=[pl.BlockSpec((RN, FH), lambda i: (i, 0))] * 2,
        out_shape=[jax.ShapeDtypeStruct((NPAD, FH), jnp.float32)] * 2,
    )(h, proj_W, proj_b, proj_ln_g, proj_ln_b)


def _layer_common(xlo, xhi, alo, ahi, deg0_ref, deg1_ref, W, b):
    i = pl.program_id(0)
    d = deg0_ref[pl.ds(i * RN, RN)] + deg1_ref[pl.ds(i * RN, RN)]
    d = d.reshape(-1, 1)
    norm = jnp.where(d > 0, 1.0 / d, 0.0)
    y = (jnp.dot(xlo, W[:FH], preferred_element_type=jnp.float32)
         + jnp.dot(xhi, W[FH:F], preferred_element_type=jnp.float32)
         + jnp.dot(alo * norm, W[F:F + FH], preferred_element_type=jnp.float32)
         + jnp.dot(ahi * norm, W[F + FH:], preferred_element_type=jnp.float32)
         + b)
    return y


def _layer_body(xlo_ref, xhi_ref, alo_ref, ahi_ref, deg0_ref, deg1_ref, W_ref, b_ref,
                g_ref, bb_ref, ylo_ref, yhi_ref):
    y = _layer_common(xlo_ref[...], xhi_ref[...], alo_ref[...], ahi_ref[...],
                      deg0_ref, deg1_ref, W_ref[...], b_ref[...])
    y = jax.nn.relu(_ln(y, g_ref[...], bb_ref[...]))
    ylo_ref[...] = y[:, :FH]
    yhi_ref[...] = y[:, FH:]


def _layer(xlo, xhi, alo, ahi, deg0, deg1, W, b, g, bb):
    return pl.pallas_call(
        _layer_body,
        grid=(NPAD // RN,),
        in_specs=[
            pl.BlockSpec((RN, FH), lambda i: (i, 0)),
            pl.BlockSpec((RN, FH), lambda i: (i, 0)),
            pl.BlockSpec((RN, FH), lambda i: (i, 0)),
            pl.BlockSpec((RN, FH), lambda i: (i, 0)),
            pl.BlockSpec((NPAD,), lambda i: (0,)),
            pl.BlockSpec((NPAD,), lambda i: (0,)),
            pl.BlockSpec((2 * F, F), lambda i: (0, 0)),
            pl.BlockSpec((1, F), lambda i: (0, 0)),
            pl.BlockSpec((1, F), lambda i: (0, 0)),
            pl.BlockSpec((1, F), lambda i: (0, 0)),
        ],
        out_specs=[pl.BlockSpec((RN, FH), lambda i: (i, 0))] * 2,
        out_shape=[jax.ShapeDtypeStruct((NPAD, FH), jnp.float32)] * 2,
    )(xlo, xhi, alo, ahi, deg0, deg1, W, b, g, bb)


def _layer_ab_body(xlo_ref, xhi_ref, alo_ref, ahi_ref, deg0_ref, deg1_ref, W_ref, b_ref,
                   g_ref, bb_ref, W1_ref, b1_ref, A_ref, B_ref):
    y = _layer_common(xlo_ref[...], xhi_ref[...], alo_ref[...], ahi_ref[...],
                      deg0_ref, deg1_ref, W_ref[...], b_ref[...])
    y = jax.nn.relu(_ln(y, g_ref[...], bb_ref[...]))
    A_ref[...] = jnp.dot(y, W1_ref[:F], preferred_element_type=jnp.float32)
    B_ref[...] = jnp.dot(y, W1_ref[F:], preferred_element_type=jnp.float32) + b1_ref[...]


def _layer_ab(xlo, xhi, alo, ahi, deg0, deg1, W, b, g, bb, W1, b1):
    return pl.pallas_call(
        _layer_ab_body,
        grid=(NPAD // RN,),
        in_specs=[
            pl.BlockSpec((RN, FH), lambda i: (i, 0)),
            pl.BlockSpec((RN, FH), lambda i: (i, 0)),
            pl.BlockSpec((RN, FH), lambda i: (i, 0)),
            pl.BlockSpec((RN, FH), lambda i: (i, 0)),
            pl.BlockSpec((NPAD,), lambda i: (0,)),
            pl.BlockSpec((NPAD,), lambda i: (0,)),
            pl.BlockSpec((2 * F, F), lambda i: (0, 0)),
            pl.BlockSpec((1, F), lambda i: (0, 0)),
            pl.BlockSpec((1, F), lambda i: (0, 0)),
            pl.BlockSpec((1, F), lambda i: (0, 0)),
            pl.BlockSpec((2 * F, F), lambda i: (0, 0)),
            pl.BlockSpec((1, F), lambda i: (0, 0)),
        ],
        out_specs=[pl.BlockSpec((RN, F), lambda i: (i, 0))] * 2,
        out_shape=[jax.ShapeDtypeStruct((NPAD, F), jnp.float32)] * 2,
    )(xlo, xhi, alo, ahi, deg0, deg1, W, b, g, bb, W1, b1)


QSCALE = 2048.0


def _final_body(e_ref, ge_ref, go_ref, be_ref, bo_ref, W2e_ref, W2o_ref, b2_ref, o_ref):
    eps = 1e-5
    e16 = pltpu.bitcast(e_ref[...], jnp.int16)
    ef = (e16.astype(jnp.float32) * (1.0 / QSCALE)).reshape(RE, 2, FH)
    e0 = ef[:, 0, :]
    e1 = ef[:, 1, :]
    m = (jnp.sum(e0, axis=-1, keepdims=True) + jnp.sum(e1, axis=-1, keepdims=True)) / F
    v = (jnp.sum((e0 - m) ** 2, axis=-1, keepdims=True)
         + jnp.sum((e1 - m) ** 2, axis=-1, keepdims=True)) / F
    rstd = lax.rsqrt(v + eps)
    y0 = jax.nn.relu((e0 - m) * rstd * ge_ref[...] + be_ref[...])
    y1 = jax.nn.relu((e1 - m) * rstd * go_ref[...] + bo_ref[...])
    o_ref[...] = (jnp.dot(y0, W2e_ref[...], preferred_element_type=jnp.float32)
                  + jnp.dot(y1, W2o_ref[...], preferred_element_type=jnp.float32)
                  + b2_ref[...])


def _final(e_pre, ge, go, be, bo, W2e, W2o, b2, ecount):
    return pl.pallas_call(
        _final_body,
        grid=(ecount // RE,),
        in_specs=[
            pl.BlockSpec((RE, FH), lambda i: (i, 0)),
            pl.BlockSpec((1, FH), lambda i: (0, 0)),
            pl.BlockSpec((1, FH), lambda i: (0, 0)),
            pl.BlockSpec((1, FH), lambda i: (0, 0)),
            pl.BlockSpec((1, FH), lambda i: (0, 0)),
            pl.BlockSpec((FH, CLASSES), lambda i: (0, 0)),
            pl.BlockSpec((FH, CLASSES), lambda i: (0, 0)),
            pl.BlockSpec((1, CLASSES), lambda i: (0, 0)),
        ],
        out_specs=pl.BlockSpec((RE, CLASSES), lambda i: (i, 0)),
        out_shape=jax.ShapeDtypeStruct((ecount, CLASSES), jnp.float32),
    )(e_pre, ge, go, be, bo, W2e, W2o, b2)


@functools.lru_cache(maxsize=None)
def _mesh():
    return plsc.VectorSubcoreMesh(core_axis_name="c", subcore_axis_name="s")


@functools.lru_cache(maxsize=None)
def _make_agg():
    def body(xlo, xhi, src1, dst3, agglo, agghi, acc, srcv, dstv,
             rows_a, rows_b, sem_a, sem_b):
        cid = lax.axis_index("c")
        sid = lax.axis_index("s")
        r0 = sid * ROWS_PER_SUB

        z16 = jnp.zeros((16,), jnp.float32)

        def zb(i, c):
            rows_a[i // 8, pl.ds((i % 8) * 16, 16)] = z16
            return c
        lax.fori_loop(0, AGG_CHUNK * 8, zb, 0)

        for j in range(ROWS_PER_SUB // AGG_CHUNK):
            pltpu.sync_copy(rows_a, acc.at[pl.ds(r0 + j * AGG_CHUNK, AGG_CHUNK)])

        eper = E // NSUB
        pltpu.sync_copy(src1.at[pl.ds(sid * eper, eper)], srcv)
        pltpu.sync_copy(dst3.at[sid], dstv)

        plsc.subcore_barrier()

        def sidx(i):
            return srcv.at[pl.ds(i * AGG_CHUNK, AGG_CHUNK)]

        def run(xref):
            pltpu.async_copy(xref.at[sidx(0)], rows_a, sem_a)

            def pair(j, c):
                ia = 2 * j
                ib = 2 * j + 1
                pltpu.async_copy(xref.at[sidx(ib)], rows_b, sem_b)
                pltpu.make_async_copy(xref.at[sidx(ia)], rows_a, sem_a).wait()
                pltpu.sync_copy(rows_a, acc.at[dstv.at[ia]], add=True)
                pltpu.async_copy(xref.at[sidx(ib + 1)], rows_a, sem_a)
                pltpu.make_async_copy(xref.at[sidx(ib)], rows_b, sem_b).wait()
                pltpu.sync_copy(rows_b, acc.at[dstv.at[ib]], add=True)
                return c
            lax.fori_loop(0, (AGG_NCH - 1) // 2, pair, 0)

            last = AGG_NCH - 1
            pltpu.make_async_copy(xref.at[sidx(last)], rows_a, sem_a).wait()
            pltpu.sync_copy(rows_a, acc.at[dstv.at[last]], add=True)

        @pl.when(cid == 0)
        def _():
            run(xlo)

        @pl.when(cid == 1)
        def _():
            run(xhi)

        plsc.subcore_barrier()

        for j in range(ROWS_PER_SUB // AGG_CHUNK):
            sl = pl.ds(r0 + j * AGG_CHUNK, AGG_CHUNK)

            @pl.when(cid == 0)
            def _():
                pltpu.sync_copy(acc.at[sl], rows_a)
                pltpu.sync_copy(rows_a, agglo.at[sl])

            @pl.when(cid == 1)
            def _():
                pltpu.sync_copy(acc.at[sl], rows_a)
                pltpu.sync_copy(rows_a, agghi.at[sl])

    return pl.kernel(
        body,
        out_type=(jax.ShapeDtypeStruct((NPAD, FH), jnp.float32),
                  jax.ShapeDtypeStruct((NPAD, FH), jnp.float32)),
        mesh=_mesh(),
        scratch_types=[
            pltpu.VMEM_SHARED((NPAD, FH), jnp.float32),
            pltpu.VMEM((E // NSUB,), jnp.int32),
            pltpu.VMEM((AGG_NCH, AGG_CHUNK), jnp.int32),
            pltpu.VMEM((AGG_CHUNK, FH), jnp.float32),
            pltpu.VMEM((AGG_CHUNK, FH), jnp.float32),
            pltpu.SemaphoreType.DMA,
            pltpu.SemaphoreType.DMA,
        ],
    )


DEG_CHUNK = 1000


@functools.lru_cache(maxsize=None)
def _make_deg():
    def body(dst, deg0, deg1, dacc, dstv, obuf, sem):
        cid = lax.axis_index("c")
        sid = lax.axis_index("s")
        r0 = sid * ROWS_PER_SUB

        z16 = jnp.zeros((16,), jnp.float32)
        o16 = jnp.ones((16,), jnp.float32)

        def zb(i, c):
            obuf[pl.ds(i * 16, 16)] = z16
            return c
        lax.fori_loop(0, ROWS_PER_SUB // 16, zb, 0)
        pltpu.sync_copy(obuf.at[pl.ds(0, ROWS_PER_SUB)], dacc.at[pl.ds(r0, ROWS_PER_SUB)])

        def ob(i, c):
            obuf[pl.ds(i * 16, 16)] = o16
            return c
        lax.fori_loop(0, (DEG_CHUNK + 15) // 16, ob, 0)

        plsc.subcore_barrier()

        eper = E // (NSUB * NCORE)
        wid = sid * NCORE + cid
        def chunk(i, c):
            b = wid * eper + i * DEG_CHUNK
            pltpu.sync_copy(dst.at[pl.ds(b, DEG_CHUNK)], dstv)
            pltpu.sync_copy(obuf.at[pl.ds(0, DEG_CHUNK)], dacc.at[dstv], add=True)
            return c
        lax.fori_loop(0, eper // DEG_CHUNK, chunk, 0)

        plsc.subcore_barrier()

        pltpu.sync_copy(dacc.at[pl.ds(r0, ROWS_PER_SUB)], obuf.at[pl.ds(0, ROWS_PER_SUB)])

        @pl.when(cid == 0)
        def _():
            pltpu.sync_copy(obuf.at[pl.ds(0, ROWS_PER_SUB)], deg0.at[pl.ds(r0, ROWS_PER_SUB)])

        @pl.when(cid == 1)
        def _():
            pltpu.sync_copy(obuf.at[pl.ds(0, ROWS_PER_SUB)], deg1.at[pl.ds(r0, ROWS_PER_SUB)])

    return pl.kernel(
        body,
        out_type=(jax.ShapeDtypeStruct((NPAD,), jnp.float32),
                  jax.ShapeDtypeStruct((NPAD,), jnp.float32)),
        mesh=_mesh(),
        scratch_types=[
            pltpu.VMEM_SHARED((NPAD,), jnp.float32),
            pltpu.VMEM((DEG_CHUNK,), jnp.int32),
            pltpu.VMEM((((DEG_CHUNK + 15) // 16) * 16,), jnp.float32),
            pltpu.SemaphoreType.DMA,
        ],
    )


@functools.lru_cache(maxsize=None)
def _make_edge(ecount, ch):
    eper = ecount // (NSUB * NCORE)
    nch = eper // ch
    assert nch % 2 == 1 and ch % 8 == 0 and ch <= 128

    def body(A, B, src1, dst1, out, srcv, dstv, a1, a2, b1_, b2_, ebuf, sem_a, sem_b):
        cid = lax.axis_index("c")
        sid = lax.axis_index("s")
        wid = sid * NCORE + cid
        base0 = wid * eper

        pltpu.sync_copy(src1.at[pl.ds(base0, eper)], srcv)
        pltpu.sync_copy(dst1.at[pl.ds(base0, eper)], dstv)

        def fire(i, bufA, bufB, sem):
            pltpu.async_copy(A.at[srcv.at[pl.ds(i * ch, ch)]], bufA, sem)
            pltpu.async_copy(B.at[dstv.at[pl.ds(i * ch, ch)]], bufB, sem)

        def stage(i, bufA, bufB, sem):
            pltpu.make_async_copy(A.at[srcv.at[pl.ds(i * ch, ch)]], bufA, sem).wait()
            pltpu.make_async_copy(B.at[dstv.at[pl.ds(i * ch, ch)]], bufB, sem).wait()

            def addrow(r, c2):
                H = jnp.full((16,), -2147450880, jnp.int32)
                L = jnp.full((16,), 2147450879, jnp.int32)
                for cc in range(FH // 16):
                    s = pl.ds(cc * 16, 16)
                    a = bufA[r, s]
                    b = bufB[r, s]
                    lo = lax.bitwise_and(a, L) + lax.bitwise_and(b, L)
                    ebuf[r, s] = lax.bitwise_xor(lo, lax.bitwise_and(lax.bitwise_xor(a, b), H))
                return c2
            lax.fori_loop(0, ch, addrow, 0)
            pltpu.sync_copy(ebuf, out.at[pl.ds(base0 + i * ch, ch)])

        fire(0, a1, a2, sem_a)

        def pairloop(j, c):
            ia = 2 * j
            ib = 2 * j + 1
            fire(ib, b1_, b2_, sem_b)
            stage(ia, a1, a2, sem_a)
            fire(ib + 1, a1, a2, sem_a)
            stage(ib, b1_, b2_, sem_b)
            return c
        lax.fori_loop(0, (nch - 1) // 2, pairloop, 0)

        stage(nch - 1, a1, a2, sem_a)

    return pl.kernel(
        body,
        out_type=jax.ShapeDtypeStruct((ecount, FH), jnp.int32),
        mesh=_mesh(),
        scratch_types=[
            pltpu.VMEM((eper,), jnp.int32),
            pltpu.VMEM((eper,), jnp.int32),
            pltpu.VMEM((ch, FH), jnp.int32),
            pltpu.VMEM((ch, FH), jnp.int32),
            pltpu.VMEM((ch, FH), jnp.int32),
            pltpu.VMEM((ch, FH), jnp.int32),
            pltpu.VMEM((ch, FH), jnp.int32),
            pltpu.SemaphoreType.DMA,
            pltpu.SemaphoreType.DMA,
        ],
    )


def kernel(h, edge_index, proj_W, proj_b, proj_ln_g, proj_ln_b,
           mp_W, mp_b, mp_ln_g, mp_ln_b, W1, b1, ln_g, ln_b, W2, b2):
    src = edge_index[0]
    dst = edge_index[1]

    deg0, deg1 = _make_deg()(dst)
    h_pad = jnp.pad(h, ((0, NPAD - N), (0, 0)))
    xlo, xhi = _proj(h_pad, proj_W, proj_b, proj_ln_g, proj_ln_b)
    dst3 = dst.reshape(NSUB, AGG_NCH, AGG_CHUNK)

    agglo, agghi = _make_agg()(xlo, xhi, src, dst3)
    ylo, yhi = _layer(xlo, xhi, agglo, agghi, deg0, deg1,
                      mp_W[0], mp_b[0].reshape(1, F),
                      mp_ln_g[0].reshape(1, F), mp_ln_b[0].reshape(1, F))
    agglo2, agghi2 = _make_agg()(ylo, yhi, src, dst3)
    A, Bm = _layer_ab(ylo, yhi, agglo2, agghi2, deg0, deg1,
                      mp_W[1], mp_b[1].reshape(1, F),
                      mp_ln_g[1].reshape(1, F), mp_ln_b[1].reshape(1, F),
                      W1, b1.reshape(1, F))
    Aq = jnp.round(A * 2048.0).astype(jnp.int16)
    Bq = jnp.round(Bm * 2048.0).astype(jnp.int16)
    Ai = lax.bitcast_convert_type(Aq.reshape(NPAD, FH, 2), jnp.int32)
    Bi = lax.bitcast_convert_type(Bq.reshape(NPAD, FH, 2), jnp.int32)
    ge = ln_g[0::2].reshape(1, FH)
    go = ln_g[1::2].reshape(1, FH)
    be = ln_b[0::2].reshape(1, FH)
    bo = ln_b[1::2].reshape(1, FH)
    W2e = W2[0::2, :]
    W2o = W2[1::2, :]
    outs = []
    off = 0
    for ecount, ch in E_SPLITS:
        s1 = lax.slice_in_dim(src, off, off + ecount)
        d1 = lax.slice_in_dim(dst, off, off + ecount)
        e_pre = _make_edge(ecount, ch)(Ai, Bi, s1, d1)
        outs.append(_final(e_pre, ge, go, be, bo, W2e, W2o,
                           b2.reshape(1, CLASSES), ecount))
        off += ecount
    return jnp.concatenate(outs, axis=0)


# speedup vs baseline: 1.1199x; 1.1199x over previous
import functools

import jax
import jax.numpy as jnp
from jax import lax
from jax.experimental import pallas as pl
from jax.experimental.pallas import tpu as pltpu
from jax.experimental.pallas import tpu_sc as plsc

N = 10000
E = 160000
F = 256
FH = 128
CLASSES = 2
DEGW = 16

NSUB = 16
NCORE = 2
NPAD = 10240
ROWS_PER_SUB = NPAD // NSUB
AGG_CHUNK = 80
AGG_NCH = E // (NSUB * AGG_CHUNK)
E_SPLITS = ((64000, 80), (96000, 40))

RN = 1024
RE = 2000


def _ln(y, g, b, eps=1e-5):
    m = jnp.mean(y, axis=-1, keepdims=True)
    v = jnp.mean((y - m) ** 2, axis=-1, keepdims=True)
    return (y - m) * lax.rsqrt(v + eps) * g + b


def _proj_body(h_ref, W_ref, b_ref, g_ref, bb_ref, xlo_ref, xhi_ref):
    y0 = jnp.dot(h_ref[:, :FH], W_ref[0], preferred_element_type=jnp.float32) + b_ref[0]
    y1 = jnp.dot(h_ref[:, FH:], W_ref[1], preferred_element_type=jnp.float32) + b_ref[1]
    xlo_ref[...] = jax.nn.relu(_ln(y0, g_ref[0], bb_ref[0]))
    xhi_ref[...] = jax.nn.relu(_ln(y1, g_ref[1], bb_ref[1]))


def _proj(h, proj_W, proj_b, proj_ln_g, proj_ln_b):
    return pl.pallas_call(
        _proj_body,
        grid=(NPAD // RN,),
        in_specs=[
            pl.BlockSpec((RN, F), lambda i: (i, 0)),
            pl.BlockSpec((2, FH, FH), lambda i: (0, 0, 0)),
            pl.BlockSpec((2, FH), lambda i: (0, 0)),
            pl.BlockSpec((2, FH), lambda i: (0, 0)),
            pl.BlockSpec((2, FH), lambda i: (0, 0)),
        ],
        out_specs=[pl.BlockSpec((RN, FH), lambda i: (i, 0))] * 2,
        out_shape=[jax.ShapeDtypeStruct((NPAD, FH), jnp.float32)] * 2,
    )(h, proj_W, proj_b, proj_ln_g, proj_ln_b)


def _layer_common(xlo, xhi, alo, ahi, deg0_ref, deg1_ref, W, b):
    i = pl.program_id(0)
    d = deg0_ref[pl.ds(i * RN, RN)] + deg1_ref[pl.ds(i * RN, RN)]
    d = d.reshape(-1, 1)
    norm = jnp.where(d > 0, 1.0 / d, 0.0)
    y = (jnp.dot(xlo, W[:FH], preferred_element_type=jnp.float32)
         + jnp.dot(xhi, W[FH:F], preferred_element_type=jnp.float32)
         + jnp.dot(alo * norm, W[F:F + FH], preferred_element_type=jnp.float32)
         + jnp.dot(ahi * norm, W[F + FH:], preferred_element_type=jnp.float32)
         + b)
    return y


def _layer_body(xlo_ref, xhi_ref, alo_ref, ahi_ref, deg0_ref, deg1_ref, W_ref, b_ref,
                g_ref, bb_ref, ylo_ref, yhi_ref):
    y = _layer_common(xlo_ref[...], xhi_ref[...], alo_ref[...], ahi_ref[...],
                      deg0_ref, deg1_ref, W_ref[...], b_ref[...])
    y = jax.nn.relu(_ln(y, g_ref[...], bb_ref[...]))
    ylo_ref[...] = y[:, :FH]
    yhi_ref[...] = y[:, FH:]


def _layer(xlo, xhi, alo, ahi, deg0, deg1, W, b, g, bb):
    return pl.pallas_call(
        _layer_body,
        grid=(NPAD // RN,),
        in_specs=[
            pl.BlockSpec((RN, FH), lambda i: (i, 0)),
            pl.BlockSpec((RN, FH), lambda i: (i, 0)),
            pl.BlockSpec((RN, FH), lambda i: (i, 0)),
            pl.BlockSpec((RN, FH), lambda i: (i, 0)),
            pl.BlockSpec((NPAD,), lambda i: (0,)),
            pl.BlockSpec((NPAD,), lambda i: (0,)),
            pl.BlockSpec((2 * F, F), lambda i: (0, 0)),
            pl.BlockSpec((1, F), lambda i: (0, 0)),
            pl.BlockSpec((1, F), lambda i: (0, 0)),
            pl.BlockSpec((1, F), lambda i: (0, 0)),
        ],
        out_specs=[pl.BlockSpec((RN, FH), lambda i: (i, 0))] * 2,
        out_shape=[jax.ShapeDtypeStruct((NPAD, FH), jnp.float32)] * 2,
    )(xlo, xhi, alo, ahi, deg0, deg1, W, b, g, bb)


def _layer_ab_body(xlo_ref, xhi_ref, alo_ref, ahi_ref, deg0_ref, deg1_ref, W_ref, b_ref,
                   g_ref, bb_ref, W1_ref, b1_ref, A_ref, B_ref):
    y = _layer_common(xlo_ref[...], xhi_ref[...], alo_ref[...], ahi_ref[...],
                      deg0_ref, deg1_ref, W_ref[...], b_ref[...])
    y = jax.nn.relu(_ln(y, g_ref[...], bb_ref[...]))
    Af = jnp.dot(y, W1_ref[:F], preferred_element_type=jnp.float32)
    Bf = jnp.dot(y, W1_ref[F:], preferred_element_type=jnp.float32) + b1_ref[...]

    def pack(x):
        q = jnp.round(x * QSCALE).astype(jnp.int32)
        lo = lax.bitwise_and(q[:, :FH], jnp.int32(0xFFFF))
        hi = lax.shift_left(q[:, FH:], jnp.int32(16))
        return lax.bitwise_or(lo, hi)

    A_ref[...] = pack(Af)
    B_ref[...] = pack(Bf)


def _layer_ab(xlo, xhi, alo, ahi, deg0, deg1, W, b, g, bb, W1, b1):
    return pl.pallas_call(
        _layer_ab_body,
        grid=(NPAD // RN,),
        in_specs=[
            pl.BlockSpec((RN, FH), lambda i: (i, 0)),
            pl.BlockSpec((RN, FH), lambda i: (i, 0)),
            pl.BlockSpec((RN, FH), lambda i: (i, 0)),
            pl.BlockSpec((RN, FH), lambda i: (i, 0)),
            pl.BlockSpec((NPAD,), lambda i: (0,)),
            pl.BlockSpec((NPAD,), lambda i: (0,)),
            pl.BlockSpec((2 * F, F), lambda i: (0, 0)),
            pl.BlockSpec((1, F), lambda i: (0, 0)),
            pl.BlockSpec((1, F), lambda i: (0, 0)),
            pl.BlockSpec((1, F), lambda i: (0, 0)),
            pl.BlockSpec((2 * F, F), lambda i: (0, 0)),
            pl.BlockSpec((1, F), lambda i: (0, 0)),
        ],
        out_specs=[pl.BlockSpec((RN, FH), lambda i: (i, 0))] * 2,
        out_shape=[jax.ShapeDtypeStruct((NPAD, FH), jnp.int32)] * 2,
    )(xlo, xhi, alo, ahi, deg0, deg1, W, b, g, bb, W1, b1)


QSCALE = 2048.0


def _final_body(e_ref, ge_ref, go_ref, be_ref, bo_ref, W2e_ref, W2o_ref, b2_ref, o_ref):
    eps = 1e-5
    e16 = pltpu.bitcast(e_ref[...], jnp.int16)
    ef = (e16.astype(jnp.float32) * (1.0 / QSCALE)).reshape(RE, 2, FH)
    e0 = ef[:, 0, :]
    e1 = ef[:, 1, :]
    m = (jnp.sum(e0, axis=-1, keepdims=True) + jnp.sum(e1, axis=-1, keepdims=True)) / F
    v = (jnp.sum((e0 - m) ** 2, axis=-1, keepdims=True)
         + jnp.sum((e1 - m) ** 2, axis=-1, keepdims=True)) / F
    rstd = lax.rsqrt(v + eps)
    y0 = jax.nn.relu((e0 - m) * rstd * ge_ref[...] + be_ref[...])
    y1 = jax.nn.relu((e1 - m) * rstd * go_ref[...] + bo_ref[...])
    o_ref[...] = (jnp.dot(y0, W2e_ref[...], preferred_element_type=jnp.float32)
                  + jnp.dot(y1, W2o_ref[...], preferred_element_type=jnp.float32)
                  + b2_ref[...])


def _final(e_pre, ge, go, be, bo, W2e, W2o, b2, ecount):
    return pl.pallas_call(
        _final_body,
        grid=(ecount // RE,),
        in_specs=[
            pl.BlockSpec((RE, FH), lambda i: (i, 0)),
            pl.BlockSpec((1, FH), lambda i: (0, 0)),
            pl.BlockSpec((1, FH), lambda i: (0, 0)),
            pl.BlockSpec((1, FH), lambda i: (0, 0)),
            pl.BlockSpec((1, FH), lambda i: (0, 0)),
            pl.BlockSpec((FH, CLASSES), lambda i: (0, 0)),
            pl.BlockSpec((FH, CLASSES), lambda i: (0, 0)),
            pl.BlockSpec((1, CLASSES), lambda i: (0, 0)),
        ],
        out_specs=pl.BlockSpec((RE, CLASSES), lambda i: (i, 0)),
        out_shape=jax.ShapeDtypeStruct((ecount, CLASSES), jnp.float32),
    )(e_pre, ge, go, be, bo, W2e, W2o, b2)


@functools.lru_cache(maxsize=None)
def _mesh():
    return plsc.VectorSubcoreMesh(core_axis_name="c", subcore_axis_name="s")


@functools.lru_cache(maxsize=None)
def _make_agg():
    def body(xlo, xhi, src1, dst3, agglo, agghi, acc, srcv, dstv,
             rows_a, rows_b, sem_a, sem_b):
        cid = lax.axis_index("c")
        sid = lax.axis_index("s")
        r0 = sid * ROWS_PER_SUB

        z16 = jnp.zeros((16,), jnp.float32)

        def zb(i, c):
            rows_a[i // 8, pl.ds((i % 8) * 16, 16)] = z16
            return c
        lax.fori_loop(0, AGG_CHUNK * 8, zb, 0)

        for j in range(ROWS_PER_SUB // AGG_CHUNK):
            pltpu.sync_copy(rows_a, acc.at[pl.ds(r0 + j * AGG_CHUNK, AGG_CHUNK)])

        eper = E // NSUB
        pltpu.sync_copy(src1.at[pl.ds(sid * eper, eper)], srcv)
        pltpu.sync_copy(dst3.at[sid], dstv)

        plsc.subcore_barrier()

        def sidx(i):
            return srcv.at[pl.ds(i * AGG_CHUNK, AGG_CHUNK)]

        def run(xref):
            pltpu.async_copy(xref.at[sidx(0)], rows_a, sem_a)

            def pair(j, c):
                ia = 2 * j
                ib = 2 * j + 1
                pltpu.async_copy(xref.at[sidx(ib)], rows_b, sem_b)
                pltpu.make_async_copy(xref.at[sidx(ia)], rows_a, sem_a).wait()
                pltpu.sync_copy(rows_a, acc.at[dstv.at[ia]], add=True)
                pltpu.async_copy(xref.at[sidx(ib + 1)], rows_a, sem_a)
                pltpu.make_async_copy(xref.at[sidx(ib)], rows_b, sem_b).wait()
                pltpu.sync_copy(rows_b, acc.at[dstv.at[ib]], add=True)
                return c
            lax.fori_loop(0, (AGG_NCH - 1) // 2, pair, 0)

            last = AGG_NCH - 1
            pltpu.make_async_copy(xref.at[sidx(last)], rows_a, sem_a).wait()
            pltpu.sync_copy(rows_a, acc.at[dstv.at[last]], add=True)

        @pl.when(cid == 0)
        def _():
            run(xlo)

        @pl.when(cid == 1)
        def _():
            run(xhi)

        plsc.subcore_barrier()

        for j in range(ROWS_PER_SUB // AGG_CHUNK):
            sl = pl.ds(r0 + j * AGG_CHUNK, AGG_CHUNK)

            @pl.when(cid == 0)
            def _():
                pltpu.sync_copy(acc.at[sl], rows_a)
                pltpu.sync_copy(rows_a, agglo.at[sl])

            @pl.when(cid == 1)
            def _():
                pltpu.sync_copy(acc.at[sl], rows_a)
                pltpu.sync_copy(rows_a, agghi.at[sl])

    return pl.kernel(
        body,
        out_type=(jax.ShapeDtypeStruct((NPAD, FH), jnp.float32),
                  jax.ShapeDtypeStruct((NPAD, FH), jnp.float32)),
        mesh=_mesh(),
        scratch_types=[
            pltpu.VMEM_SHARED((NPAD, FH), jnp.float32),
            pltpu.VMEM((E // NSUB,), jnp.int32),
            pltpu.VMEM((AGG_NCH, AGG_CHUNK), jnp.int32),
            pltpu.VMEM((AGG_CHUNK, FH), jnp.float32),
            pltpu.VMEM((AGG_CHUNK, FH), jnp.float32),
            pltpu.SemaphoreType.DMA,
            pltpu.SemaphoreType.DMA,
        ],
    )


DEG_CHUNK = 1000


@functools.lru_cache(maxsize=None)
def _make_deg():
    def body(dst, deg0, deg1, dacc, dstv, obuf, sem):
        cid = lax.axis_index("c")
        sid = lax.axis_index("s")
        r0 = sid * ROWS_PER_SUB

        z16 = jnp.zeros((16,), jnp.float32)
        o16 = jnp.ones((16,), jnp.float32)

        def zb(i, c):
            obuf[pl.ds(i * 16, 16)] = z16
            return c
        lax.fori_loop(0, ROWS_PER_SUB // 16, zb, 0)
        pltpu.sync_copy(obuf.at[pl.ds(0, ROWS_PER_SUB)], dacc.at[pl.ds(r0, ROWS_PER_SUB)])

        def ob(i, c):
            obuf[pl.ds(i * 16, 16)] = o16
            return c
        lax.fori_loop(0, (DEG_CHUNK + 15) // 16, ob, 0)

        plsc.subcore_barrier()

        eper = E // (NSUB * NCORE)
        wid = sid * NCORE + cid
        def chunk(i, c):
            b = wid * eper + i * DEG_CHUNK
            pltpu.sync_copy(dst.at[pl.ds(b, DEG_CHUNK)], dstv)
            pltpu.sync_copy(obuf.at[pl.ds(0, DEG_CHUNK)], dacc.at[dstv], add=True)
            return c
        lax.fori_loop(0, eper // DEG_CHUNK, chunk, 0)

        plsc.subcore_barrier()

        pltpu.sync_copy(dacc.at[pl.ds(r0, ROWS_PER_SUB)], obuf.at[pl.ds(0, ROWS_PER_SUB)])

        @pl.when(cid == 0)
        def _():
            pltpu.sync_copy(obuf.at[pl.ds(0, ROWS_PER_SUB)], deg0.at[pl.ds(r0, ROWS_PER_SUB)])

        @pl.when(cid == 1)
        def _():
            pltpu.sync_copy(obuf.at[pl.ds(0, ROWS_PER_SUB)], deg1.at[pl.ds(r0, ROWS_PER_SUB)])

    return pl.kernel(
        body,
        out_type=(jax.ShapeDtypeStruct((NPAD,), jnp.float32),
                  jax.ShapeDtypeStruct((NPAD,), jnp.float32)),
        mesh=_mesh(),
        scratch_types=[
            pltpu.VMEM_SHARED((NPAD,), jnp.float32),
            pltpu.VMEM((DEG_CHUNK,), jnp.int32),
            pltpu.VMEM((((DEG_CHUNK + 15) // 16) * 16,), jnp.float32),
            pltpu.SemaphoreType.DMA,
        ],
    )


@functools.lru_cache(maxsize=None)
def _make_edge(ecount, ch):
    eper = ecount // (NSUB * NCORE)
    nch = eper // ch
    assert nch % 2 == 1 and ch % 8 == 0 and ch <= 128

    def body(A, B, src1, dst1, out, srcv, dstv, a1, a2, b1_, b2_, ebuf, sem_a, sem_b):
        cid = lax.axis_index("c")
        sid = lax.axis_index("s")
        wid = sid * NCORE + cid
        base0 = wid * eper

        pltpu.sync_copy(src1.at[pl.ds(base0, eper)], srcv)
        pltpu.sync_copy(dst1.at[pl.ds(base0, eper)], dstv)

        def fire(i, bufA, bufB, sem):
            pltpu.async_copy(A.at[srcv.at[pl.ds(i * ch, ch)]], bufA, sem)
            pltpu.async_copy(B.at[dstv.at[pl.ds(i * ch, ch)]], bufB, sem)

        def stage(i, bufA, bufB, sem):
            pltpu.make_async_copy(A.at[srcv.at[pl.ds(i * ch, ch)]], bufA, sem).wait()
            pltpu.make_async_copy(B.at[dstv.at[pl.ds(i * ch, ch)]], bufB, sem).wait()

            def addrow(r, c2):
                H = jnp.full((16,), -2147450880, jnp.int32)
                L = jnp.full((16,), 2147450879, jnp.int32)
                for cc in range(FH // 16):
                    s = pl.ds(cc * 16, 16)
                    a = bufA[r, s]
                    b = bufB[r, s]
                    lo = lax.bitwise_and(a, L) + lax.bitwise_and(b, L)
                    ebuf[r, s] = lax.bitwise_xor(lo, lax.bitwise_and(lax.bitwise_xor(a, b), H))
                return c2
            lax.fori_loop(0, ch, addrow, 0)
            pltpu.sync_copy(ebuf, out.at[pl.ds(base0 + i * ch, ch)])

        fire(0, a1, a2, sem_a)

        def pairloop(j, c):
            ia = 2 * j
            ib = 2 * j + 1
            fire(ib, b1_, b2_, sem_b)
            stage(ia, a1, a2, sem_a)
            fire(ib + 1, a1, a2, sem_a)
            stage(ib, b1_, b2_, sem_b)
            return c
        lax.fori_loop(0, (nch - 1) // 2, pairloop, 0)

        stage(nch - 1, a1, a2, sem_a)

    return pl.kernel(
        body,
        out_type=jax.ShapeDtypeStruct((ecount, FH), jnp.int32),
        mesh=_mesh(),
        scratch_types=[
            pltpu.VMEM((eper,), jnp.int32),
            pltpu.VMEM((eper,), jnp.int32),
            pltpu.VMEM((ch, FH), jnp.int32),
            pltpu.VMEM((ch, FH), jnp.int32),
            pltpu.VMEM((ch, FH), jnp.int32),
            pltpu.VMEM((ch, FH), jnp.int32),
            pltpu.VMEM((ch, FH), jnp.int32),
            pltpu.SemaphoreType.DMA,
            pltpu.SemaphoreType.DMA,
        ],
    )


def kernel(h, edge_index, proj_W, proj_b, proj_ln_g, proj_ln_b,
           mp_W, mp_b, mp_ln_g, mp_ln_b, W1, b1, ln_g, ln_b, W2, b2):
    src = edge_index[0]
    dst = edge_index[1]

    deg0, deg1 = _make_deg()(dst)
    h_pad = jnp.pad(h, ((0, NPAD - N), (0, 0)))
    xlo, xhi = _proj(h_pad, proj_W, proj_b, proj_ln_g, proj_ln_b)
    dst3 = dst.reshape(NSUB, AGG_NCH, AGG_CHUNK)

    agglo, agghi = _make_agg()(xlo, xhi, src, dst3)
    ylo, yhi = _layer(xlo, xhi, agglo, agghi, deg0, deg1,
                      mp_W[0], mp_b[0].reshape(1, F),
                      mp_ln_g[0].reshape(1, F), mp_ln_b[0].reshape(1, F))
    agglo2, agghi2 = _make_agg()(ylo, yhi, src, dst3)
    A, Bm = _layer_ab(ylo, yhi, agglo2, agghi2, deg0, deg1,
                      mp_W[1], mp_b[1].reshape(1, F),
                      mp_ln_g[1].reshape(1, F), mp_ln_b[1].reshape(1, F),
                      W1, b1.reshape(1, F))
    Ai = A
    Bi = Bm
    ge = ln_g[:FH].reshape(1, FH)
    go = ln_g[FH:].reshape(1, FH)
    be = ln_b[:FH].reshape(1, FH)
    bo = ln_b[FH:].reshape(1, FH)
    W2e = W2[:FH, :]
    W2o = W2[FH:, :]
    outs = []
    off = 0
    for ecount, ch in E_SPLITS:
        s1 = lax.slice_in_dim(src, off, off + ecount)
        d1 = lax.slice_in_dim(dst, off, off + ecount)
        e_pre = _make_edge(ecount, ch)(Ai, Bi, s1, d1)
        outs.append(_final(e_pre, ge, go, be, bo, W2e, W2o,
                           b2.reshape(1, CLASSES), ecount))
        off += ecount
    return jnp.concatenate(outs, axis=0)


# speedup vs baseline: 2.0854x; 1.8621x over previous
import functools

import jax
import jax.numpy as jnp
from jax import lax
from jax.experimental import pallas as pl
from jax.experimental.pallas import tpu as pltpu
from jax.experimental.pallas import tpu_sc as plsc

N = 10000
E = 160000
F = 256
FH = 128
CLASSES = 2
DEGW = 16

NSUB = 16
NCORE = 2
NPAD = 10240
ROWS_PER_SUB = NPAD // NSUB
AGG_CHUNK = 80
AGG_NCH = E // (NSUB * AGG_CHUNK)
E_SPLITS = ((64000, 80), (96000, 40))

RN = 1024
RE = 2000


def _ln(y, g, b, eps=1e-5):
    m = jnp.mean(y, axis=-1, keepdims=True)
    v = jnp.mean((y - m) ** 2, axis=-1, keepdims=True)
    return (y - m) * lax.rsqrt(v + eps) * g + b


def _proj_body(h_ref, W_ref, b_ref, g_ref, bb_ref, xlo_ref, xhi_ref):
    y0 = jnp.dot(h_ref[:, :FH], W_ref[0], preferred_element_type=jnp.float32) + b_ref[0]
    y1 = jnp.dot(h_ref[:, FH:], W_ref[1], preferred_element_type=jnp.float32) + b_ref[1]
    xlo_ref[...] = jax.nn.relu(_ln(y0, g_ref[0], bb_ref[0]))
    xhi_ref[...] = jax.nn.relu(_ln(y1, g_ref[1], bb_ref[1]))


def _proj(h, proj_W, proj_b, proj_ln_g, proj_ln_b):
    return pl.pallas_call(
        _proj_body,
        grid=(NPAD // RN,),
        in_specs=[
            pl.BlockSpec((RN, F), lambda i: (i, 0)),
            pl.BlockSpec((2, FH, FH), lambda i: (0, 0, 0)),
            pl.BlockSpec((2, FH), lambda i: (0, 0)),
            pl.BlockSpec((2, FH), lambda i: (0, 0)),
            pl.BlockSpec((2, FH), lambda i: (0, 0)),
        ],
        out_specs=[pl.BlockSpec((RN, FH), lambda i: (i, 0))] * 2,
        out_shape=[jax.ShapeDtypeStruct((NPAD, FH), jnp.float32)] * 2,
    )(h, proj_W, proj_b, proj_ln_g, proj_ln_b)


def _layer_common(xlo, xhi, alo, ahi, deg0_ref, deg1_ref, W, b):
    i = pl.program_id(0)
    d = deg0_ref[pl.ds(i * RN, RN)] + deg1_ref[pl.ds(i * RN, RN)]
    d = d.reshape(-1, 1)
    norm = jnp.where(d > 0, 1.0 / d, 0.0)
    y = (jnp.dot(xlo, W[:FH], preferred_element_type=jnp.float32)
         + jnp.dot(xhi, W[FH:F], preferred_element_type=jnp.float32)
         + jnp.dot(alo * norm, W[F:F + FH], preferred_element_type=jnp.float32)
         + jnp.dot(ahi * norm, W[F + FH:], preferred_element_type=jnp.float32)
         + b)
    return y


def _layer_body(xlo_ref, xhi_ref, alo_ref, ahi_ref, deg0_ref, deg1_ref, W_ref, b_ref,
                g_ref, bb_ref, ylo_ref, yhi_ref):
    y = _layer_common(xlo_ref[...], xhi_ref[...], alo_ref[...], ahi_ref[...],
                      deg0_ref, deg1_ref, W_ref[...], b_ref[...])
    y = jax.nn.relu(_ln(y, g_ref[...], bb_ref[...]))
    ylo_ref[...] = y[:, :FH]
    yhi_ref[...] = y[:, FH:]


def _layer(xlo, xhi, alo, ahi, deg0, deg1, W, b, g, bb):
    return pl.pallas_call(
        _layer_body,
        grid=(NPAD // RN,),
        in_specs=[
            pl.BlockSpec((RN, FH), lambda i: (i, 0)),
            pl.BlockSpec((RN, FH), lambda i: (i, 0)),
            pl.BlockSpec((RN, FH), lambda i: (i, 0)),
            pl.BlockSpec((RN, FH), lambda i: (i, 0)),
            pl.BlockSpec((NPAD,), lambda i: (0,)),
            pl.BlockSpec((NPAD,), lambda i: (0,)),
            pl.BlockSpec((2 * F, F), lambda i: (0, 0)),
            pl.BlockSpec((1, F), lambda i: (0, 0)),
            pl.BlockSpec((1, F), lambda i: (0, 0)),
            pl.BlockSpec((1, F), lambda i: (0, 0)),
        ],
        out_specs=[pl.BlockSpec((RN, FH), lambda i: (i, 0))] * 2,
        out_shape=[jax.ShapeDtypeStruct((NPAD, FH), jnp.float32)] * 2,
    )(xlo, xhi, alo, ahi, deg0, deg1, W, b, g, bb)


def _layer_ab_body(xlo_ref, xhi_ref, alo_ref, ahi_ref, deg0_ref, deg1_ref, W_ref, b_ref,
                   g_ref, bb_ref, W1_ref, b1_ref, A_ref, B_ref):
    y = _layer_common(xlo_ref[...], xhi_ref[...], alo_ref[...], ahi_ref[...],
                      deg0_ref, deg1_ref, W_ref[...], b_ref[...])
    y = jax.nn.relu(_ln(y, g_ref[...], bb_ref[...]))
    Af = jnp.dot(y, W1_ref[:F], preferred_element_type=jnp.float32)
    Bf = jnp.dot(y, W1_ref[F:], preferred_element_type=jnp.float32) + b1_ref[...]

    def pack(x):
        q = jnp.round(x * QSCALE).astype(jnp.int32)
        lo = lax.bitwise_and(q[:, :FH], jnp.int32(0xFFFF))
        hi = lax.shift_left(q[:, FH:], jnp.int32(16))
        return lax.bitwise_or(lo, hi)

    A_ref[...] = pack(Af)
    B_ref[...] = pack(Bf)


def _layer_ab(xlo, xhi, alo, ahi, deg0, deg1, W, b, g, bb, W1, b1):
    return pl.pallas_call(
        _layer_ab_body,
        grid=(NPAD // RN,),
        in_specs=[
            pl.BlockSpec((RN, FH), lambda i: (i, 0)),
            pl.BlockSpec((RN, FH), lambda i: (i, 0)),
            pl.BlockSpec((RN, FH), lambda i: (i, 0)),
            pl.BlockSpec((RN, FH), lambda i: (i, 0)),
            pl.BlockSpec((NPAD,), lambda i: (0,)),
            pl.BlockSpec((NPAD,), lambda i: (0,)),
            pl.BlockSpec((2 * F, F), lambda i: (0, 0)),
            pl.BlockSpec((1, F), lambda i: (0, 0)),
            pl.BlockSpec((1, F), lambda i: (0, 0)),
            pl.BlockSpec((1, F), lambda i: (0, 0)),
            pl.BlockSpec((2 * F, F), lambda i: (0, 0)),
            pl.BlockSpec((1, F), lambda i: (0, 0)),
        ],
        out_specs=[pl.BlockSpec((RN, FH), lambda i: (i, 0))] * 2,
        out_shape=[jax.ShapeDtypeStruct((NPAD, FH), jnp.int32)] * 2,
    )(xlo, xhi, alo, ahi, deg0, deg1, W, b, g, bb, W1, b1)


QSCALE = 2048.0


def _final_body(e_ref, ge_ref, go_ref, be_ref, bo_ref, W2e_ref, W2o_ref, b2_ref, o_ref):
    eps = 1e-5
    x = e_ref[...]
    e0 = lax.shift_right_arithmetic(lax.shift_left(x, 16), 16).astype(jnp.float32) * (1.0 / QSCALE)
    e1 = lax.shift_right_arithmetic(x, 16).astype(jnp.float32) * (1.0 / QSCALE)
    m = (jnp.sum(e0, axis=-1, keepdims=True) + jnp.sum(e1, axis=-1, keepdims=True)) / F
    v = (jnp.sum((e0 - m) ** 2, axis=-1, keepdims=True)
         + jnp.sum((e1 - m) ** 2, axis=-1, keepdims=True)) / F
    rstd = lax.rsqrt(v + eps)
    y0 = jax.nn.relu((e0 - m) * rstd * ge_ref[...] + be_ref[...])
    y1 = jax.nn.relu((e1 - m) * rstd * go_ref[...] + bo_ref[...])
    o_ref[...] = (jnp.dot(y0, W2e_ref[...], preferred_element_type=jnp.float32)
                  + jnp.dot(y1, W2o_ref[...], preferred_element_type=jnp.float32)
                  + b2_ref[...])


def _final(e_pre, ge, go, be, bo, W2e, W2o, b2, ecount):
    return pl.pallas_call(
        _final_body,
        grid=(ecount // RE,),
        in_specs=[
            pl.BlockSpec((RE, FH), lambda i: (i, 0)),
            pl.BlockSpec((1, FH), lambda i: (0, 0)),
            pl.BlockSpec((1, FH), lambda i: (0, 0)),
            pl.BlockSpec((1, FH), lambda i: (0, 0)),
            pl.BlockSpec((1, FH), lambda i: (0, 0)),
            pl.BlockSpec((FH, CLASSES), lambda i: (0, 0)),
            pl.BlockSpec((FH, CLASSES), lambda i: (0, 0)),
            pl.BlockSpec((1, CLASSES), lambda i: (0, 0)),
        ],
        out_specs=pl.BlockSpec((RE, CLASSES), lambda i: (i, 0)),
        out_shape=jax.ShapeDtypeStruct((ecount, CLASSES), jnp.float32),
    )(e_pre, ge, go, be, bo, W2e, W2o, b2)


@functools.lru_cache(maxsize=None)
def _mesh():
    return plsc.VectorSubcoreMesh(core_axis_name="c", subcore_axis_name="s")


@functools.lru_cache(maxsize=None)
def _make_agg():
    def body(xlo, xhi, src1, dst3, agglo, agghi, acc, srcv, dstv,
             rows_a, rows_b, sem_a, sem_b):
        cid = lax.axis_index("c")
        sid = lax.axis_index("s")
        r0 = sid * ROWS_PER_SUB

        z16 = jnp.zeros((16,), jnp.float32)

        def zb(i, c):
            rows_a[i // 8, pl.ds((i % 8) * 16, 16)] = z16
            return c
        lax.fori_loop(0, AGG_CHUNK * 8, zb, 0)

        for j in range(ROWS_PER_SUB // AGG_CHUNK):
            pltpu.sync_copy(rows_a, acc.at[pl.ds(r0 + j * AGG_CHUNK, AGG_CHUNK)])

        eper = E // NSUB
        pltpu.sync_copy(src1.at[pl.ds(sid * eper, eper)], srcv)
        pltpu.sync_copy(dst3.at[sid], dstv)

        plsc.subcore_barrier()

        def sidx(i):
            return srcv.at[pl.ds(i * AGG_CHUNK, AGG_CHUNK)]

        def run(xref):
            pltpu.async_copy(xref.at[sidx(0)], rows_a, sem_a)

            def pair(j, c):
                ia = 2 * j
                ib = 2 * j + 1
                pltpu.async_copy(xref.at[sidx(ib)], rows_b, sem_b)
                pltpu.make_async_copy(xref.at[sidx(ia)], rows_a, sem_a).wait()
                pltpu.sync_copy(rows_a, acc.at[dstv.at[ia]], add=True)
                pltpu.async_copy(xref.at[sidx(ib + 1)], rows_a, sem_a)
                pltpu.make_async_copy(xref.at[sidx(ib)], rows_b, sem_b).wait()
                pltpu.sync_copy(rows_b, acc.at[dstv.at[ib]], add=True)
                return c
            lax.fori_loop(0, (AGG_NCH - 1) // 2, pair, 0)

            last = AGG_NCH - 1
            pltpu.make_async_copy(xref.at[sidx(last)], rows_a, sem_a).wait()
            pltpu.sync_copy(rows_a, acc.at[dstv.at[last]], add=True)

        @pl.when(cid == 0)
        def _():
            run(xlo)

        @pl.when(cid == 1)
        def _():
            run(xhi)

        plsc.subcore_barrier()

        for j in range(ROWS_PER_SUB // AGG_CHUNK):
            sl = pl.ds(r0 + j * AGG_CHUNK, AGG_CHUNK)

            @pl.when(cid == 0)
            def _():
                pltpu.sync_copy(acc.at[sl], rows_a)
                pltpu.sync_copy(rows_a, agglo.at[sl])

            @pl.when(cid == 1)
            def _():
                pltpu.sync_copy(acc.at[sl], rows_a)
                pltpu.sync_copy(rows_a, agghi.at[sl])

    return pl.kernel(
        body,
        out_type=(jax.ShapeDtypeStruct((NPAD, FH), jnp.float32),
                  jax.ShapeDtypeStruct((NPAD, FH), jnp.float32)),
        mesh=_mesh(),
        scratch_types=[
            pltpu.VMEM_SHARED((NPAD, FH), jnp.float32),
            pltpu.VMEM((E // NSUB,), jnp.int32),
            pltpu.VMEM((AGG_NCH, AGG_CHUNK), jnp.int32),
            pltpu.VMEM((AGG_CHUNK, FH), jnp.float32),
            pltpu.VMEM((AGG_CHUNK, FH), jnp.float32),
            pltpu.SemaphoreType.DMA,
            pltpu.SemaphoreType.DMA,
        ],
    )


DEG_CHUNK = 1000


@functools.lru_cache(maxsize=None)
def _make_deg():
    def body(dst, deg0, deg1, dacc, dstv, obuf, sem):
        cid = lax.axis_index("c")
        sid = lax.axis_index("s")
        r0 = sid * ROWS_PER_SUB

        z16 = jnp.zeros((16,), jnp.float32)
        o16 = jnp.ones((16,), jnp.float32)

        def zb(i, c):
            obuf[pl.ds(i * 16, 16)] = z16
            return c
        lax.fori_loop(0, ROWS_PER_SUB // 16, zb, 0)
        pltpu.sync_copy(obuf.at[pl.ds(0, ROWS_PER_SUB)], dacc.at[pl.ds(r0, ROWS_PER_SUB)])

        def ob(i, c):
            obuf[pl.ds(i * 16, 16)] = o16
            return c
        lax.fori_loop(0, (DEG_CHUNK + 15) // 16, ob, 0)

        plsc.subcore_barrier()

        eper = E // (NSUB * NCORE)
        wid = sid * NCORE + cid
        def chunk(i, c):
            b = wid * eper + i * DEG_CHUNK
            pltpu.sync_copy(dst.at[pl.ds(b, DEG_CHUNK)], dstv)
            pltpu.sync_copy(obuf.at[pl.ds(0, DEG_CHUNK)], dacc.at[dstv], add=True)
            return c
        lax.fori_loop(0, eper // DEG_CHUNK, chunk, 0)

        plsc.subcore_barrier()

        pltpu.sync_copy(dacc.at[pl.ds(r0, ROWS_PER_SUB)], obuf.at[pl.ds(0, ROWS_PER_SUB)])

        @pl.when(cid == 0)
        def _():
            pltpu.sync_copy(obuf.at[pl.ds(0, ROWS_PER_SUB)], deg0.at[pl.ds(r0, ROWS_PER_SUB)])

        @pl.when(cid == 1)
        def _():
            pltpu.sync_copy(obuf.at[pl.ds(0, ROWS_PER_SUB)], deg1.at[pl.ds(r0, ROWS_PER_SUB)])

    return pl.kernel(
        body,
        out_type=(jax.ShapeDtypeStruct((NPAD,), jnp.float32),
                  jax.ShapeDtypeStruct((NPAD,), jnp.float32)),
        mesh=_mesh(),
        scratch_types=[
            pltpu.VMEM_SHARED((NPAD,), jnp.float32),
            pltpu.VMEM((DEG_CHUNK,), jnp.int32),
            pltpu.VMEM((((DEG_CHUNK + 15) // 16) * 16,), jnp.float32),
            pltpu.SemaphoreType.DMA,
        ],
    )


@functools.lru_cache(maxsize=None)
def _make_edge(ecount, ch):
    eper = ecount // (NSUB * NCORE)
    nch = eper // ch
    assert nch % 2 == 1 and ch % 8 == 0 and ch <= 128

    def body(A, B, src1, dst1, out, srcv, dstv, a1, a2, b1_, b2_, ebuf, sem_a, sem_b):
        cid = lax.axis_index("c")
        sid = lax.axis_index("s")
        wid = sid * NCORE + cid
        base0 = wid * eper

        pltpu.sync_copy(src1.at[pl.ds(base0, eper)], srcv)
        pltpu.sync_copy(dst1.at[pl.ds(base0, eper)], dstv)

        def fire(i, bufA, bufB, sem):
            pltpu.async_copy(A.at[srcv.at[pl.ds(i * ch, ch)]], bufA, sem)
            pltpu.async_copy(B.at[dstv.at[pl.ds(i * ch, ch)]], bufB, sem)

        def stage(i, bufA, bufB, sem):
            pltpu.make_async_copy(A.at[srcv.at[pl.ds(i * ch, ch)]], bufA, sem).wait()
            pltpu.make_async_copy(B.at[dstv.at[pl.ds(i * ch, ch)]], bufB, sem).wait()

            def addrow(r, c2):
                H = jnp.full((16,), -2147450880, jnp.int32)
                L = jnp.full((16,), 2147450879, jnp.int32)
                for cc in range(FH // 16):
                    s = pl.ds(cc * 16, 16)
                    a = bufA[r, s]
                    b = bufB[r, s]
                    lo = lax.bitwise_and(a, L) + lax.bitwise_and(b, L)
                    ebuf[r, s] = lax.bitwise_xor(lo, lax.bitwise_and(lax.bitwise_xor(a, b), H))
                return c2
            lax.fori_loop(0, ch, addrow, 0)
            pltpu.sync_copy(ebuf, out.at[pl.ds(base0 + i * ch, ch)])

        fire(0, a1, a2, sem_a)

        def pairloop(j, c):
            ia = 2 * j
            ib = 2 * j + 1
            fire(ib, b1_, b2_, sem_b)
            stage(ia, a1, a2, sem_a)
            fire(ib + 1, a1, a2, sem_a)
            stage(ib, b1_, b2_, sem_b)
            return c
        lax.fori_loop(0, (nch - 1) // 2, pairloop, 0)

        stage(nch - 1, a1, a2, sem_a)

    return pl.kernel(
        body,
        out_type=jax.ShapeDtypeStruct((ecount, FH), jnp.int32),
        mesh=_mesh(),
        scratch_types=[
            pltpu.VMEM((eper,), jnp.int32),
            pltpu.VMEM((eper,), jnp.int32),
            pltpu.VMEM((ch, FH), jnp.int32),
            pltpu.VMEM((ch, FH), jnp.int32),
            pltpu.VMEM((ch, FH), jnp.int32),
            pltpu.VMEM((ch, FH), jnp.int32),
            pltpu.VMEM((ch, FH), jnp.int32),
            pltpu.SemaphoreType.DMA,
            pltpu.SemaphoreType.DMA,
        ],
    )


def kernel(h, edge_index, proj_W, proj_b, proj_ln_g, proj_ln_b,
           mp_W, mp_b, mp_ln_g, mp_ln_b, W1, b1, ln_g, ln_b, W2, b2):
    src = edge_index[0]
    dst = edge_index[1]

    deg0, deg1 = _make_deg()(dst)
    h_pad = jnp.pad(h, ((0, NPAD - N), (0, 0)))
    xlo, xhi = _proj(h_pad, proj_W, proj_b, proj_ln_g, proj_ln_b)
    dst3 = dst.reshape(NSUB, AGG_NCH, AGG_CHUNK)

    agglo, agghi = _make_agg()(xlo, xhi, src, dst3)
    ylo, yhi = _layer(xlo, xhi, agglo, agghi, deg0, deg1,
                      mp_W[0], mp_b[0].reshape(1, F),
                      mp_ln_g[0].reshape(1, F), mp_ln_b[0].reshape(1, F))
    agglo2, agghi2 = _make_agg()(ylo, yhi, src, dst3)
    A, Bm = _layer_ab(ylo, yhi, agglo2, agghi2, deg0, deg1,
                      mp_W[1], mp_b[1].reshape(1, F),
                      mp_ln_g[1].reshape(1, F), mp_ln_b[1].reshape(1, F),
                      W1, b1.reshape(1, F))
    Ai = A
    Bi = Bm
    ge = ln_g[:FH].reshape(1, FH)
    go = ln_g[FH:].reshape(1, FH)
    be = ln_b[:FH].reshape(1, FH)
    bo = ln_b[FH:].reshape(1, FH)
    W2e = W2[:FH, :]
    W2o = W2[FH:, :]
    outs = []
    off = 0
    for ecount, ch in E_SPLITS:
        s1 = lax.slice_in_dim(src, off, off + ecount)
        d1 = lax.slice_in_dim(dst, off, off + ecount)
        e_pre = _make_edge(ecount, ch)(Ai, Bi, s1, d1)
        outs.append(_final(e_pre, ge, go, be, bo, W2e, W2o,
                           b2.reshape(1, CLASSES), ecount))
        off += ecount
    return jnp.concatenate(outs, axis=0)
